# baseline (speedup 1.0000x reference)
import functools

import jax
import jax.numpy as jnp
from jax.experimental import pallas as pl
from jax.experimental.pallas import tpu as pltpu

_E = 8
_K = 2
_H = 64
_EH = _E * _H
_PADE = 128


def _moe_kernel(x_ref, w1_ref, wg_ref, w2_ref, out_ref, ep_ref, loss_ref,
                *, nblk, ntok, tblk):
    i = pl.program_id(0)
    xb = x_ref[...]

    hpre = jax.lax.dot_general(
        xb, w1_ref[...], (((1,), (1,)), ((), ())),
        preferred_element_type=jnp.float32)
    logits = jax.lax.dot_general(
        xb, wg_ref[...], (((1,), (1,)), ((), ())),
        preferred_element_type=jnp.float32)

    lt = jnp.transpose(logits, (1, 0))
    sub = jax.lax.broadcasted_iota(jnp.int32, lt.shape, 0)
    big = jnp.int32(_E)
    lmax = jnp.max(lt, axis=0, keepdims=True)
    el = jnp.exp(lt - lmax)
    s = jnp.sum(el, axis=0, keepdims=True)
    rinv = 1.0 / s
    i1 = jnp.min(jnp.where(lt == lmax, sub, big), axis=0, keepdims=True)
    sel1 = sub == i1
    l2 = jnp.where(sel1, -jnp.inf, lt)
    lmax2 = jnp.max(l2, axis=0, keepdims=True)
    i2 = jnp.min(jnp.where(l2 == lmax2, sub, big), axis=0, keepdims=True)
    sel2 = sub == i2
    m2 = jnp.exp(lmax2 - lmax) * rinv
    cwt = jnp.where(sel1, rinv, 0.0) + jnp.where(sel2, m2, 0.0)

    gate_t = el * rinv
    ones_t = jnp.ones((tblk, 128), jnp.float32)
    ep_blk = jnp.dot(gate_t, ones_t,
                     preferred_element_type=jnp.float32)

    @pl.when(i == 0)
    def _():
        ep_ref[...] = jnp.zeros_like(ep_ref)

    ep_ref[...] += ep_blk

    rep = (jax.lax.broadcasted_iota(jnp.int32, (_E, _EH), 1) // _H
           == jax.lax.broadcasted_iota(jnp.int32, (_E, _EH), 0)
           ).astype(jnp.float32)
    scale = jax.lax.dot_general(
        cwt, rep, (((0,), (0,)), ((), ())),
        preferred_element_type=jnp.float32)

    h = hpre * jax.nn.sigmoid(hpre)
    out_ref[...] = jnp.dot(h * scale, w2_ref[...],
                           preferred_element_type=jnp.float32)

    @pl.when(i == nblk - 1)
    def _():
        ep = ep_ref[...] / ntok
        m = jnp.mean(ep, axis=0, keepdims=True)
        var = jnp.mean((ep - m) ** 2, axis=0, keepdims=True)
        loss_ref[...] = (var / (m * m + 1e-10))[:, 0:1]


def kernel(x, Wg, W1, W2):
    B, S, D = x.shape
    N = B * S
    T = 512
    nblk = N // T

    xf = x.reshape(N, D)
    w1_r = W1.reshape(_EH, D)
    w2_r = jnp.transpose(W2, (0, 2, 1)).reshape(_EH, D)

    out, _, loss = pl.pallas_call(
        functools.partial(_moe_kernel, nblk=nblk, ntok=N, tblk=T),
        grid=(nblk,),
        in_specs=[
            pl.BlockSpec((T, D), lambda i: (i, 0)),
            pl.BlockSpec((_EH, D), lambda i: (0, 0)),
            pl.BlockSpec((_E, D), lambda i: (0, 0)),
            pl.BlockSpec((_EH, D), lambda i: (0, 0)),
        ],
        out_specs=[
            pl.BlockSpec((T, D), lambda i: (i, 0)),
            pl.BlockSpec((_E, 128), lambda i: (0, 0)),
            pl.BlockSpec((1, 1), lambda i: (0, 0)),
        ],
        out_shape=[
            jax.ShapeDtypeStruct((N, D), jnp.float32),
            jax.ShapeDtypeStruct((_E, 128), jnp.float32),
            jax.ShapeDtypeStruct((1, 1), jnp.float32),
        ],
        compiler_params=pltpu.CompilerParams(
            dimension_semantics=("arbitrary",)),
    )(xf, w1_r, Wg, w2_r)

    return out.reshape(B, S, D), loss[0, 0]


# speedup vs baseline: 1.1356x; 1.1356x over previous
import functools

import jax
import jax.numpy as jnp
from jax.experimental import pallas as pl
from jax.experimental.pallas import tpu as pltpu

_E = 8
_K = 2
_H = 64
_EH = _E * _H
_PADE = 128


def _moe_kernel(x_ref, w1_ref, wg_ref, w2_ref, out_ref, ep_ref, loss_ref,
                *, nblk, ntok, tblk):
    i = pl.program_id(0)
    xb = x_ref[...]

    logits = jax.lax.dot_general(
        xb, wg_ref[...], (((1,), (1,)), ((), ())),
        preferred_element_type=jnp.float32)
    hpre = jax.lax.dot_general(
        xb, w1_ref[...], (((1,), (1,)), ((), ())),
        preferred_element_type=jnp.float32)

    lt = jnp.transpose(logits, (1, 0))
    sub = jax.lax.broadcasted_iota(jnp.int32, lt.shape, 0)
    big = jnp.int32(_E)
    lmax = jnp.max(lt, axis=0, keepdims=True)
    el = jnp.exp(lt - lmax)
    s = jnp.sum(el, axis=0, keepdims=True)
    rinv = 1.0 / s
    i1 = jnp.min(jnp.where(lt == lmax, sub, big), axis=0, keepdims=True)
    sel1 = sub == i1
    l2 = jnp.where(sel1, -jnp.inf, lt)
    lmax2 = jnp.max(l2, axis=0, keepdims=True)
    i2 = jnp.min(jnp.where(l2 == lmax2, sub, big), axis=0, keepdims=True)
    sel2 = sub == i2
    m2 = jnp.exp(lmax2 - lmax) * rinv
    cwt = jnp.where(sel1, rinv, 0.0) + jnp.where(sel2, m2, 0.0)

    gate_t = el * rinv
    ones_t = jnp.ones((tblk, 128), jnp.float32)
    ep_blk = jnp.dot(gate_t, ones_t,
                     preferred_element_type=jnp.float32)

    @pl.when(i == 0)
    def _():
        ep_ref[...] = jnp.zeros_like(ep_ref)

    ep_ref[...] += ep_blk

    rep = (jax.lax.broadcasted_iota(jnp.int32, (_E, _EH), 1) // _H
           == jax.lax.broadcasted_iota(jnp.int32, (_E, _EH), 0)
           ).astype(jnp.float32)
    scale = jax.lax.dot_general(
        cwt, rep, (((0,), (0,)), ((), ())),
        preferred_element_type=jnp.float32)

    h = hpre * jax.nn.sigmoid(hpre)
    out_ref[...] = jnp.dot(h * scale, w2_ref[...],
                           preferred_element_type=jnp.float32)

    @pl.when(i == nblk - 1)
    def _():
        ep = ep_ref[...] / ntok
        m = jnp.mean(ep, axis=0, keepdims=True)
        var = jnp.mean((ep - m) ** 2, axis=0, keepdims=True)
        loss_ref[...] = (var / (m * m + 1e-10))[:, 0:1]


def kernel(x, Wg, W1, W2):
    B, S, D = x.shape
    N = B * S
    T = 1024
    nblk = N // T

    xf = x.reshape(N, D)
    w1_r = W1.reshape(_EH, D)
    w2_r = jnp.transpose(W2, (0, 2, 1)).reshape(_EH, D)

    out, _, loss = pl.pallas_call(
        functools.partial(_moe_kernel, nblk=nblk, ntok=N, tblk=T),
        grid=(nblk,),
        in_specs=[
            pl.BlockSpec((T, D), lambda i: (i, 0)),
            pl.BlockSpec((_EH, D), lambda i: (0, 0)),
            pl.BlockSpec((_E, D), lambda i: (0, 0)),
            pl.BlockSpec((_EH, D), lambda i: (0, 0)),
        ],
        out_specs=[
            pl.BlockSpec((T, D), lambda i: (i, 0)),
            pl.BlockSpec((_E, 128), lambda i: (0, 0)),
            pl.BlockSpec((1, 1), lambda i: (0, 0)),
        ],
        out_shape=[
            jax.ShapeDtypeStruct((N, D), jnp.float32),
            jax.ShapeDtypeStruct((_E, 128), jnp.float32),
            jax.ShapeDtypeStruct((1, 1), jnp.float32),
        ],
        compiler_params=pltpu.CompilerParams(
            dimension_semantics=("arbitrary",)),
    )(xf, w1_r, Wg, w2_r)

    return out.reshape(B, S, D), loss[0, 0]
